# Initial kernel scaffold; baseline (speedup 1.0000x reference)
import jax
import jax.numpy as jnp
from jax.experimental import pallas as pl


def kernel(pred, target, score):
    raise NotImplementedError("write your pallas kernel here")


# speedup vs baseline: 2.0783x; 2.0783x over previous
import functools

import jax
import jax.numpy as jnp
from jax import lax
from jax.experimental import pallas as pl
from jax.experimental.pallas import tpu as pltpu
from jax.experimental.pallas import tpu_sc as plsc

_N, _C = 100000, 80
_NW = 32
_RPW = 3136
_NP = _NW * _RPW
_J = 28
_M = 112
_K = _M // 16

_TC_G, _TC_BLK = 25, 2500


def _tc_dense_body(p_ref, out_ref):
    i = pl.program_id(0)
    x = p_ref[...]
    t = jnp.exp(-jnp.abs(x))
    ell = jnp.log(1.0 + t)
    sig = jnp.where(x >= 0.0, 1.0, t) / (1.0 + t)
    base = (jnp.maximum(x, 0.0) + ell) * sig * sig

    @pl.when(i == 0)
    def _():
        out_ref[...] = jnp.zeros((1, 128), jnp.float32)

    out_ref[...] += jnp.sum(base, axis=(0, 1)).reshape(1, 128)


_tc_dense = pl.pallas_call(
    _tc_dense_body,
    grid=(_TC_G,),
    in_specs=[pl.BlockSpec((1, _TC_BLK, 128), lambda i: (i, 0, 0))],
    out_specs=pl.BlockSpec((1, 128), lambda i: (0, 0)),
    out_shape=jax.ShapeDtypeStruct((1, 128), jnp.float32),
)


def _sc_mesh():
    return plsc.VectorSubcoreMesh(core_axis_name="c", subcore_axis_name="s")


@functools.partial(
    pl.kernel,
    mesh=_sc_mesh(),
    out_type=jax.ShapeDtypeStruct((_NW, 16), jnp.float32),
    scratch_types=[
        pltpu.VMEM((_RPW,), jnp.int32),
        pltpu.VMEM((_RPW,), jnp.float32),
        pltpu.VMEM((_J, _M), jnp.int32),
        pltpu.VMEM((_J, _M), jnp.float32),
        pltpu.VMEM((16,), jnp.float32),
        pltpu.SemaphoreType.DMA,
    ],
)
def _sc_corr(pred_hbm, tgt_hbm, scr_hbm, out_hbm,
             tgt_v, scr_v, idx_v, gat_v, acc_v, sem):
    cid = lax.axis_index("c")
    sid = lax.axis_index("s")
    wid = sid * 2 + cid
    row0 = wid * _RPW
    pltpu.sync_copy(tgt_hbm.at[pl.ds(row0, _RPW)], tgt_v)
    pltpu.sync_copy(scr_hbm.at[pl.ds(row0, _RPW)], scr_v)
    lane = lax.broadcasted_iota(jnp.int32, (16,), 0)

    def build_and_fire(j, carry):
        for k in range(_K):
            off = j * _M + k * 16
            tgt = tgt_v[pl.ds(off, 16)]
            lbl = tgt - 1
            pos = lbl >= 0
            row = row0 + off + lane
            idx = jnp.where(pos, row * _C + lbl, 0)
            idx_v[j, pl.ds(k * 16, 16)] = idx
        pltpu.async_copy(pred_hbm.at[idx_v.at[j]], gat_v.at[j], sem)
        return carry

    lax.fori_loop(0, _J, build_and_fire, 0)

    def drain(j, carry):
        pltpu.make_async_copy(pred_hbm.at[idx_v.at[j]], gat_v.at[j], sem).wait()
        return carry

    lax.fori_loop(0, _J, drain, 0)

    def accumulate(j, acc):
        for k in range(_K):
            off = j * _M + k * 16
            tgt = tgt_v[pl.ds(off, 16)]
            s = scr_v[pl.ds(off, 16)]
            x = gat_v[j, pl.ds(k * 16, 16)]
            pos = (tgt - 1) >= 0
            t = jnp.exp(-jnp.abs(x))
            v = t / (2.0 + t)
            v2 = v * v
            poly = 1.0 + v2 * (1.0 / 3.0 + v2 * (1.0 / 5.0 + v2 * (
                1.0 / 7.0 + v2 * (1.0 / 9.0 + v2 * (1.0 / 11.0 + v2 * (1.0 / 13.0))))))
            ell = 2.0 * v * poly
            sig = jnp.where(x >= 0.0, 1.0, t) / (1.0 + t)
            b0 = jnp.maximum(x, 0.0) + ell
            base = b0 * sig * sig
            pt = s - sig
            new = (b0 - x * s) * pt * pt
            acc = acc + jnp.where(pos, new - base, 0.0)
        return acc

    acc = lax.fori_loop(0, _J, accumulate, jnp.zeros((16,), jnp.float32))
    acc_v[...] = acc
    pltpu.sync_copy(acc_v, out_hbm.at[wid])


def kernel(pred, target, score):
    tgt = target.astype(jnp.int32)
    scr = score.astype(jnp.float32)
    pad = _NP - _N
    tgt_p = jnp.concatenate([tgt, jnp.zeros((pad,), jnp.int32)])
    scr_p = jnp.concatenate([scr, jnp.zeros((pad,), jnp.float32)])
    pred_flat = pred.reshape(-1)

    corr_parts = _sc_corr(pred_flat, tgt_p, scr_p)
    dense = _tc_dense(pred_flat.reshape(_TC_G, _TC_BLK, 128))

    total = jnp.sum(dense) + jnp.sum(corr_parts)
    return total / jnp.float32(_N * _C)


# speedup vs baseline: 3.2902x; 1.5831x over previous
import functools

import jax
import jax.numpy as jnp
from jax import lax
from jax.experimental import pallas as pl
from jax.experimental.pallas import tpu as pltpu
from jax.experimental.pallas import tpu_sc as plsc

_N, _C = 100000, 80
_NW = 32
_RPW = 3136
_NP = _NW * _RPW

_TC_BLK = 4000
_TC_G = _N // _TC_BLK


def _tc_body(p_ref, t_ref, sum_ref, pp_ref):
    i = pl.program_id(0)
    x = p_ref[...]
    t = jnp.exp(-jnp.abs(x))
    ell = jnp.log(1.0 + t)
    sig = jnp.where(x >= 0.0, 1.0, t) / (1.0 + t)
    base = (jnp.maximum(x, 0.0) + ell) * sig * sig

    @pl.when(i == 0)
    def _():
        sum_ref[...] = jnp.zeros((1, _C), jnp.float32)

    sum_ref[...] += jnp.sum(base, axis=0, keepdims=True)

    tgt = t_ref[0, 0, :]
    bc = jnp.clip(tgt - 1, 0, _C - 1)
    onehot = lax.broadcasted_iota(jnp.int32, (_TC_BLK, _C), 1) == bc[:, None]
    pp_ref[0, 0, :] = jnp.sum(jnp.where(onehot, x, 0.0), axis=1)


_tc_dense = pl.pallas_call(
    _tc_body,
    grid=(_TC_G,),
    in_specs=[
        pl.BlockSpec((_TC_BLK, _C), lambda i: (i, 0)),
        pl.BlockSpec((1, 1, _TC_BLK), lambda i: (i, 0, 0)),
    ],
    out_specs=[
        pl.BlockSpec((1, _C), lambda i: (0, 0)),
        pl.BlockSpec((1, 1, _TC_BLK), lambda i: (i, 0, 0)),
    ],
    out_shape=[
        jax.ShapeDtypeStruct((1, _C), jnp.float32),
        jax.ShapeDtypeStruct((_TC_G, 1, _TC_BLK), jnp.float32),
    ],
)


def _sc_mesh():
    return plsc.VectorSubcoreMesh(core_axis_name="c", subcore_axis_name="s")


@functools.partial(
    pl.kernel,
    mesh=_sc_mesh(),
    out_type=jax.ShapeDtypeStruct((_NW, 16), jnp.float32),
    scratch_types=[
        pltpu.VMEM((_RPW,), jnp.float32),
        pltpu.VMEM((_RPW,), jnp.int32),
        pltpu.VMEM((_RPW,), jnp.float32),
        pltpu.VMEM((16,), jnp.float32),
    ],
)
def _sc_corr(pp_hbm, tgt_hbm, scr_hbm, out_hbm, pp_v, tgt_v, scr_v, acc_v):
    cid = lax.axis_index("c")
    sid = lax.axis_index("s")
    wid = sid * 2 + cid
    row0 = wid * _RPW
    pltpu.sync_copy(pp_hbm.at[pl.ds(row0, _RPW)], pp_v)
    pltpu.sync_copy(tgt_hbm.at[pl.ds(row0, _RPW)], tgt_v)
    pltpu.sync_copy(scr_hbm.at[pl.ds(row0, _RPW)], scr_v)

    def accumulate(i, acc):
        off = i * 16
        tgt = tgt_v[pl.ds(off, 16)]
        s = scr_v[pl.ds(off, 16)]
        x = pp_v[pl.ds(off, 16)]
        pos = (tgt - 1) >= 0
        t = jnp.exp(-jnp.abs(x))
        v = t / (2.0 + t)
        v2 = v * v
        poly = 1.0 + v2 * (1.0 / 3.0 + v2 * (1.0 / 5.0 + v2 * (
            1.0 / 7.0 + v2 * (1.0 / 9.0 + v2 * (1.0 / 11.0 + v2 * (1.0 / 13.0))))))
        ell = 2.0 * v * poly
        sig = jnp.where(x >= 0.0, 1.0, t) / (1.0 + t)
        b0 = jnp.maximum(x, 0.0) + ell
        base = b0 * sig * sig
        pt = s - sig
        new = (b0 - x * s) * pt * pt
        return acc + jnp.where(pos, new - base, 0.0)

    acc = lax.fori_loop(0, _RPW // 16, accumulate, jnp.zeros((16,), jnp.float32))
    acc_v[...] = acc
    pltpu.sync_copy(acc_v, out_hbm.at[wid])


def kernel(pred, target, score):
    tgt = target.astype(jnp.int32)
    scr = score.astype(jnp.float32)

    sum80, pp = _tc_dense(pred, tgt.reshape(_TC_G, 1, _TC_BLK))

    pad = _NP - _N
    pp_p = jnp.concatenate([pp.reshape(-1), jnp.zeros((pad,), jnp.float32)])
    tgt_p = jnp.concatenate([tgt, jnp.zeros((pad,), jnp.int32)])
    scr_p = jnp.concatenate([scr, jnp.zeros((pad,), jnp.float32)])
    corr_parts = _sc_corr(pp_p, tgt_p, scr_p)

    total = jnp.sum(sum80) + jnp.sum(corr_parts)
    return total / jnp.float32(_N * _C)


# speedup vs baseline: 5.8502x; 1.7781x over previous
import functools

import jax
import jax.numpy as jnp
from jax import lax
from jax.experimental import pallas as pl
from jax.experimental.pallas import tpu as pltpu
from jax.experimental.pallas import tpu_sc as plsc

_N, _C = 100000, 80
_NW = 32
_RPW = 3136
_NP = _NW * _RPW

_TC_BLK = 2048
_TC_G = _NP // _TC_BLK


def _tc_body(p_ref, t_ref, sum_ref, pp_ref):
    i = pl.program_id(0)
    xt = p_ref[...].T
    th = jnp.tanh(xt * 0.5)
    sig = 0.5 + 0.5 * th
    sp = -jnp.log(0.5 - 0.5 * th)
    base = sp * sig * sig
    col = i * _TC_BLK + lax.broadcasted_iota(jnp.int32, (1, _TC_BLK), 1)
    base = jnp.where(col < _N, base, 0.0)

    @pl.when(i == 0)
    def _():
        sum_ref[...] = jnp.zeros((1, _TC_BLK), jnp.float32)

    sum_ref[...] += jnp.sum(base, axis=0, keepdims=True)

    tgt = t_ref[...]
    bc = (tgt - 1)[None, :]
    ohT = lax.broadcasted_iota(jnp.int32, (_C, _TC_BLK), 0) == bc
    pp_ref[...] = jnp.sum(jnp.where(ohT, xt, 0.0), axis=0)


_tc_dense = pl.pallas_call(
    _tc_body,
    grid=(_TC_G,),
    in_specs=[
        pl.BlockSpec((_TC_BLK, _C), lambda i: (i, 0)),
        pl.BlockSpec((_TC_BLK,), lambda i: (i,)),
    ],
    out_specs=[
        pl.BlockSpec((1, _TC_BLK), lambda i: (0, 0)),
        pl.BlockSpec((_TC_BLK,), lambda i: (i,)),
    ],
    out_shape=[
        jax.ShapeDtypeStruct((1, _TC_BLK), jnp.float32),
        jax.ShapeDtypeStruct((_NP,), jnp.float32),
    ],
)


def _sc_mesh():
    return plsc.VectorSubcoreMesh(core_axis_name="c", subcore_axis_name="s")


@functools.partial(
    pl.kernel,
    mesh=_sc_mesh(),
    out_type=jax.ShapeDtypeStruct((_NW, 16), jnp.float32),
    scratch_types=[
        pltpu.VMEM((_RPW,), jnp.float32),
        pltpu.VMEM((_RPW,), jnp.int32),
        pltpu.VMEM((_RPW,), jnp.float32),
        pltpu.VMEM((16,), jnp.float32),
    ],
)
def _sc_corr(pp_hbm, tgt_hbm, scr_hbm, out_hbm, pp_v, tgt_v, scr_v, acc_v):
    cid = lax.axis_index("c")
    sid = lax.axis_index("s")
    wid = sid * 2 + cid
    row0 = wid * _RPW
    pltpu.sync_copy(pp_hbm.at[pl.ds(row0, _RPW)], pp_v)
    pltpu.sync_copy(tgt_hbm.at[pl.ds(row0, _RPW)], tgt_v)
    pltpu.sync_copy(scr_hbm.at[pl.ds(row0, _RPW)], scr_v)

    def accumulate(i, acc):
        off = i * 16
        tgt = tgt_v[pl.ds(off, 16)]
        s = scr_v[pl.ds(off, 16)]
        x = pp_v[pl.ds(off, 16)]
        pos = (tgt - 1) >= 0
        t = jnp.exp(-jnp.abs(x))
        v = t / (2.0 + t)
        v2 = v * v
        poly = 1.0 + v2 * (1.0 / 3.0 + v2 * (1.0 / 5.0 + v2 * (
            1.0 / 7.0 + v2 * (1.0 / 9.0 + v2 * (1.0 / 11.0 + v2 * (1.0 / 13.0))))))
        ell = 2.0 * v * poly
        sig = jnp.where(x >= 0.0, 1.0, t) / (1.0 + t)
        b0 = jnp.maximum(x, 0.0) + ell
        base = b0 * sig * sig
        pt = s - sig
        new = (b0 - x * s) * pt * pt
        return acc + jnp.where(pos, new - base, 0.0)

    acc = lax.fori_loop(0, _RPW // 16, accumulate, jnp.zeros((16,), jnp.float32))
    acc_v[...] = acc
    pltpu.sync_copy(acc_v, out_hbm.at[wid])


def kernel(pred, target, score):
    tgt = target.astype(jnp.int32)
    scr = score.astype(jnp.float32)

    pad = _NP - _N
    tgt_p = jnp.concatenate([tgt, jnp.zeros((pad,), jnp.int32)])
    scr_p = jnp.concatenate([scr, jnp.zeros((pad,), jnp.float32)])

    sum80, pp_p = _tc_dense(pred, tgt_p)
    corr_parts = _sc_corr(pp_p, tgt_p, scr_p)

    total = jnp.sum(sum80) + jnp.sum(corr_parts)
    return total / jnp.float32(_N * _C)


# speedup vs baseline: 9.5216x; 1.6276x over previous
import functools

import jax
import jax.numpy as jnp
from jax import lax
from jax.experimental import pallas as pl
from jax.experimental.pallas import tpu as pltpu
from jax.experimental.pallas import tpu_sc as plsc

_N, _C = 100000, 80
_NW = 32
_RPW = 3136
_NP = _NW * _RPW

_TC_BLK = 2048
_TC_G = _NP // _TC_BLK


def _tc_body(p_ref, t_ref, sum_ref, pp_ref):
    i = pl.program_id(0)
    xt = p_ref[...]
    th = jnp.tanh(xt * 0.5)
    sig = 0.5 + 0.5 * th
    sp = -jnp.log(0.5 - 0.5 * th)
    base = sp * sig * sig
    col = i * _TC_BLK + lax.broadcasted_iota(jnp.int32, (1, _TC_BLK), 1)
    base = jnp.where(col < _N, base, 0.0)

    @pl.when(i == 0)
    def _():
        sum_ref[...] = jnp.zeros((1, _TC_BLK), jnp.float32)

    sum_ref[...] += jnp.sum(base, axis=0, keepdims=True)

    tgt = t_ref[...]
    bc = (tgt - 1)[None, :]
    ohT = lax.broadcasted_iota(jnp.int32, (_C, _TC_BLK), 0) == bc
    pp_ref[...] = jnp.sum(jnp.where(ohT, xt, 0.0), axis=0)


_tc_dense = pl.pallas_call(
    _tc_body,
    grid=(_TC_G,),
    in_specs=[
        pl.BlockSpec((_C, _TC_BLK), lambda i: (0, i)),
        pl.BlockSpec((_TC_BLK,), lambda i: (i,)),
    ],
    out_specs=[
        pl.BlockSpec((1, _TC_BLK), lambda i: (0, 0)),
        pl.BlockSpec((_TC_BLK,), lambda i: (i,)),
    ],
    out_shape=[
        jax.ShapeDtypeStruct((1, _TC_BLK), jnp.float32),
        jax.ShapeDtypeStruct((_NP,), jnp.float32),
    ],
)


def _sc_mesh():
    return plsc.VectorSubcoreMesh(core_axis_name="c", subcore_axis_name="s")


@functools.partial(
    pl.kernel,
    mesh=_sc_mesh(),
    out_type=jax.ShapeDtypeStruct((_NW, 16), jnp.float32),
    scratch_types=[
        pltpu.VMEM((_RPW,), jnp.float32),
        pltpu.VMEM((_RPW,), jnp.int32),
        pltpu.VMEM((_RPW,), jnp.float32),
        pltpu.VMEM((16,), jnp.float32),
    ],
)
def _sc_corr(pp_hbm, tgt_hbm, scr_hbm, out_hbm, pp_v, tgt_v, scr_v, acc_v):
    cid = lax.axis_index("c")
    sid = lax.axis_index("s")
    wid = sid * 2 + cid
    first = wid * _RPW
    row0 = jnp.minimum(first, _N - _RPW)
    pltpu.sync_copy(pp_hbm.at[pl.ds(row0, _RPW)], pp_v)
    pltpu.sync_copy(tgt_hbm.at[pl.ds(row0, _RPW)], tgt_v)
    pltpu.sync_copy(scr_hbm.at[pl.ds(row0, _RPW)], scr_v)
    lane = lax.broadcasted_iota(jnp.int32, (16,), 0)

    def accumulate(i, acc):
        off = i * 16
        tgt = tgt_v[pl.ds(off, 16)]
        s = scr_v[pl.ds(off, 16)]
        x = pp_v[pl.ds(off, 16)]
        pos = ((tgt - 1) >= 0) & ((row0 + off + lane) >= first)
        t = jnp.exp(-jnp.abs(x))
        v = t / (2.0 + t)
        v2 = v * v
        poly = 1.0 + v2 * (1.0 / 3.0 + v2 * (1.0 / 5.0 + v2 * (
            1.0 / 7.0 + v2 * (1.0 / 9.0 + v2 * (1.0 / 11.0 + v2 * (1.0 / 13.0))))))
        ell = 2.0 * v * poly
        sig = jnp.where(x >= 0.0, 1.0, t) / (1.0 + t)
        b0 = jnp.maximum(x, 0.0) + ell
        base = b0 * sig * sig
        pt = s - sig
        new = (b0 - x * s) * pt * pt
        return acc + jnp.where(pos, new - base, 0.0)

    acc = lax.fori_loop(0, _RPW // 16, accumulate, jnp.zeros((16,), jnp.float32))
    acc_v[...] = acc
    pltpu.sync_copy(acc_v, out_hbm.at[wid])


def kernel(pred, target, score):
    tgt = target.astype(jnp.int32)
    scr = score.astype(jnp.float32)

    sum_l, pp_p = _tc_dense(pred.T, tgt)
    corr_parts = _sc_corr(pp_p, tgt, scr)

    total = jnp.sum(sum_l) + jnp.sum(corr_parts)
    return total / jnp.float32(_N * _C)
